# Initial kernel scaffold; baseline (speedup 1.0000x reference)
import jax
import jax.numpy as jnp
from jax.experimental import pallas as pl


def kernel(x, edge_index, edge_attr, node_type, edge_type, node_emb, edge_emb, fm_w0, fm_b0, g1_w0, g1_b0, g2_w0, g2_b0, fm_w1, fm_b1, g1_w1, g1_b1, g2_w1, g2_b1):
    raise NotImplementedError("write your pallas kernel here")


# speedup vs baseline: 2.3402x; 2.3402x over previous
import functools

import jax
import jax.numpy as jnp
from jax import lax
from jax.experimental import pallas as pl
from jax.experimental.pallas import tpu as pltpu
from jax.experimental.pallas import tpu_sc as plsc

_PREC = lax.Precision.HIGHEST


def _tca_body(x_ref, fmw_ref, fmb_ref, wxi_ref, wxj_ref, oh_ref, nemb_ref,
              wnti_ref, wntj_ref, h_ref, p_ref, q_ref):
    h = jnp.dot(x_ref[...], fmw_ref[...], preferred_element_type=jnp.float32,
                precision=_PREC) + fmb_ref[...]
    h_ref[...] = h
    ti = jnp.dot(nemb_ref[...], wnti_ref[...],
                 preferred_element_type=jnp.float32, precision=_PREC)
    tj = jnp.dot(nemb_ref[...], wntj_ref[...],
                 preferred_element_type=jnp.float32, precision=_PREC)
    oh = oh_ref[...]
    p_ref[...] = (jnp.dot(h, wxi_ref[...], preferred_element_type=jnp.float32,
                          precision=_PREC)
                  + jnp.dot(oh, ti, preferred_element_type=jnp.float32,
                            precision=_PREC))
    q_ref[...] = (jnp.dot(h, wxj_ref[...], preferred_element_type=jnp.float32,
                          precision=_PREC)
                  + jnp.dot(oh, tj, preferred_element_type=jnp.float32,
                            precision=_PREC))


def _tc_a(x, fm_w, fm_b, wxi, wxj, nt_oh, node_emb, wnti, wntj, blk):
    n, f = x.shape
    o = fm_w.shape[1]
    nd = node_emb.shape[1]
    grid = n // blk
    full = lambda i: (0, 0)
    outs = [jax.ShapeDtypeStruct((n, o), jnp.float32)] * 3
    return pl.pallas_call(
        _tca_body,
        grid=(grid,),
        in_specs=[
            pl.BlockSpec((blk, f), lambda i: (i, 0)),
            pl.BlockSpec((f, o), full),
            pl.BlockSpec((1, o), full),
            pl.BlockSpec((o, o), full),
            pl.BlockSpec((o, o), full),
            pl.BlockSpec((blk, 16), lambda i: (i, 0)),
            pl.BlockSpec((16, nd), full),
            pl.BlockSpec((nd, o), full),
            pl.BlockSpec((nd, o), full),
        ],
        out_specs=[pl.BlockSpec((blk, o), lambda i: (i, 0))] * 3,
        out_shape=outs,
    )(x, fm_w, fm_b.reshape(1, o), wxi, wxj, nt_oh, node_emb, wnti, wntj)


def _tcb_body(g1_ref, g2_ref, g3_ref, eoh_ref, ew_ref, eemb_ref, wet_ref,
              g1b_ref, g2w_ref, g2b_ref, out_ref):
    et_tab = jnp.dot(eemb_ref[...], wet_ref[...],
                     preferred_element_type=jnp.float32,
                     precision=_PREC) + g1b_ref[...]
    u = g1_ref[...] + g2_ref[...] + jnp.dot(
        eoh_ref[...], et_tab, preferred_element_type=jnp.float32,
        precision=_PREC)
    u = jnp.maximum(u, 0.0)
    t = jnp.sum(u * g2w_ref[...], axis=1, keepdims=True) + g2b_ref[...]
    s = jnp.maximum(t, 0.0) * ew_ref[...]
    m = g3_ref[...] * s
    half = m.shape[1] // 2
    out_ref[0] = m[:, :half]
    out_ref[1] = m[:, half:]


def _tc_b(g1, g2, g3, et_oh, ew, edge_emb, wet, g1b, g2w, g2b, blk):
    e, o = g1.shape
    ed = edge_emb.shape[1]
    grid = e // blk
    full = lambda i: (0, 0)
    return pl.pallas_call(
        _tcb_body,
        grid=(grid,),
        in_specs=[
            pl.BlockSpec((blk, o), lambda i: (i, 0)),
            pl.BlockSpec((blk, o), lambda i: (i, 0)),
            pl.BlockSpec((blk, o), lambda i: (i, 0)),
            pl.BlockSpec((blk, 16), lambda i: (i, 0)),
            pl.BlockSpec((blk, 1), lambda i: (i, 0)),
            pl.BlockSpec((16, ed), full),
            pl.BlockSpec((ed, o), full),
            pl.BlockSpec((1, o), full),
            pl.BlockSpec((1, o), full),
            pl.BlockSpec((1, 1), full),
        ],
        out_specs=[pl.BlockSpec((2, blk, o // 2), lambda i: (0, i, 0))],
        out_shape=[jax.ShapeDtypeStruct((2, e, o // 2), jnp.float32)],
    )(g1, g2, g3, et_oh, ew, edge_emb, wet, g1b.reshape(1, o),
      g2w.reshape(1, o), g2b.reshape(1, 1))[0]


def _tcc_body(h_ref, a_ref, out_ref):
    half = h_ref.shape[1] // 2
    v0 = h_ref[:, :half] + a_ref[0]
    v1 = h_ref[:, half:] + a_ref[1]
    out_ref[:, :half] = jnp.where(v0 >= 0.0, v0, 0.01 * v0)
    out_ref[:, half:] = jnp.where(v1 >= 0.0, v1, 0.01 * v1)


def _tc_c(h, aggr, blk):
    n, o = h.shape
    grid = n // blk
    return pl.pallas_call(
        _tcc_body,
        grid=(grid,),
        in_specs=[
            pl.BlockSpec((blk, o), lambda i: (i, 0)),
            pl.BlockSpec((2, blk, o // 2), lambda i: (0, i, 0)),
        ],
        out_specs=pl.BlockSpec((blk, o), lambda i: (i, 0)),
        out_shape=jax.ShapeDtypeStruct((n, o), jnp.float32),
    )(h, aggr)


def _sc_gather(p, q, h, dst, src):
    n, d = p.shape
    e = dst.shape[0]
    nw = 32
    ch = 128
    nchunks = e // ch
    per_w = -(-nchunks // nw)
    mesh = plsc.VectorSubcoreMesh(core_axis_name="c", subcore_axis_name="s")
    outs = [jax.ShapeDtypeStruct((e, d), jnp.float32)] * 3
    scratch = [
        pltpu.VMEM((ch,), jnp.int32),
        pltpu.VMEM((ch, d), jnp.float32),
        pltpu.SemaphoreType.DMA,
    ]

    @functools.partial(pl.kernel, mesh=mesh, out_type=outs,
                       scratch_types=scratch)
    def k(p_hbm, q_hbm, h_hbm, dst_hbm, src_hbm, g1_hbm, g2_hbm, g3_hbm,
          idx_v, rows_v, sem):
        wid = lax.axis_index("s") * 2 + lax.axis_index("c")

        def gather_chunk(tbl, ihbm, ohbm, base):
            pltpu.sync_copy(ihbm.at[pl.ds(base, ch)], idx_v)
            pltpu.async_copy(tbl.at[idx_v], rows_v, sem).wait()
            pltpu.sync_copy(rows_v, ohbm.at[pl.ds(base, ch)])

        @pl.loop(0, per_w)
        def _(j):
            cidx = wid + nw * j

            @pl.when(cidx < nchunks)
            def _():
                b = cidx * ch
                gather_chunk(p_hbm, dst_hbm, g1_hbm, b)
                gather_chunk(q_hbm, src_hbm, g2_hbm, b)
                gather_chunk(h_hbm, src_hbm, g3_hbm, b)

    return k(p, q, h, dst, src)


def _sc_scatter(msg, dst2, zeros_half):
    _, e, d2 = msg.shape
    n = zeros_half.shape[0]
    w = 128
    steps = e // w
    ns = 16
    rows = n // ns
    r_lo = (rows // 8) * 8
    r_hi = n - r_lo * (ns - 1)
    mesh = plsc.VectorSubcoreMesh(core_axis_name="c", subcore_axis_name="s")

    @functools.partial(
        pl.kernel, mesh=mesh,
        out_type=jax.ShapeDtypeStruct((2, n, d2), jnp.float32),
        scratch_types=[pltpu.VMEM_SHARED((n, d2), jnp.float32)])
    def k(msg_hbm, dst_hbm, z_hbm, out_hbm, aggr_sh):
        cid = lax.axis_index("c")
        sid = lax.axis_index("s")

        @pl.when(sid < ns - 1)
        def _():
            pltpu.sync_copy(z_hbm.at[pl.ds(sid * r_lo, r_lo)],
                            aggr_sh.at[pl.ds(sid * r_lo, r_lo)])

        @pl.when(sid == ns - 1)
        def _():
            pltpu.sync_copy(z_hbm.at[pl.ds((ns - 1) * r_lo, r_hi)],
                            aggr_sh.at[pl.ds((ns - 1) * r_lo, r_hi)])

        plsc.subcore_barrier()

        def body(msg_v, idx_v):
            pltpu.sync_copy(msg_v, aggr_sh.at[idx_v.at[0]], add=True)

        pltpu.emit_pipeline(
            body,
            grid=(steps,),
            in_specs=[
                pl.BlockSpec((w, d2), lambda i: (i, 0)),
                pl.BlockSpec((1, w), lambda i: (0, i)),
            ],
            out_specs=[],
            core_axis_name="s",
            dimension_semantics=(pltpu.PARALLEL,),
        )(msg_hbm.at[cid], dst_hbm)

        plsc.subcore_barrier()

        @pl.when(sid < ns - 1)
        def _():
            pltpu.sync_copy(aggr_sh.at[pl.ds(sid * r_lo, r_lo)],
                            out_hbm.at[cid].at[pl.ds(sid * r_lo, r_lo)])

        @pl.when(sid == ns - 1)
        def _():
            pltpu.sync_copy(aggr_sh.at[pl.ds((ns - 1) * r_lo, r_hi)],
                            out_hbm.at[cid].at[pl.ds((ns - 1) * r_lo, r_hi)])

    return k(msg, dst2, zeros_half)


def kernel(x, edge_index, edge_attr, node_type, edge_type, node_emb, edge_emb,
           fm_w0, fm_b0, g1_w0, g1_b0, g2_w0, g2_b0,
           fm_w1, fm_b1, g1_w1, g1_b1, g2_w1, g2_b1):
    n, f = x.shape[1], x.shape[2]
    e = edge_index.shape[1]
    o = fm_w0.shape[1]
    nd = node_emb.shape[1]

    xs = x.reshape(n, f)
    src = edge_index[0]
    dst = edge_index[1]
    nt_oh = jax.nn.one_hot(node_type, 16, dtype=jnp.float32)
    et_oh = jax.nn.one_hot(edge_type, 16, dtype=jnp.float32)
    ew = edge_attr.reshape(e, 1)
    dst2 = dst.reshape(1, e)
    zeros_half = jnp.zeros((n, o // 2), jnp.float32)

    def layer(h_in, fm_w, fm_b, g1w, g1b, g2w, g2b):
        wxi = g1w[0:o]
        wxj = g1w[o:2 * o]
        wnti = g1w[2 * o:2 * o + nd]
        wntj = g1w[2 * o + nd:2 * o + 2 * nd]
        wet = g1w[2 * o + 2 * nd:]
        h, p, q = _tc_a(h_in, fm_w, fm_b, wxi, wxj, nt_oh, node_emb,
                        wnti, wntj, blk=1000)
        g1, g2, g3 = _sc_gather(p, q, h, dst, src)
        msg = _tc_b(g1, g2, g3, et_oh, ew, edge_emb, wet, g1b, g2w, g2b,
                    blk=2000)
        aggr = _sc_scatter(msg, dst2, zeros_half)
        return _tc_c(h, aggr, blk=1000)

    h1 = layer(xs, fm_w0, fm_b0, g1_w0, g1_b0, g2_w0, g2_b0)
    h2 = layer(h1, fm_w1, fm_b1, g1_w1, g1_b1, g2_w1, g2_b1)
    return h2.reshape(1, n, o)


# speedup vs baseline: 2.6407x; 1.1284x over previous
import functools

import jax
import jax.numpy as jnp
from jax import lax
from jax.experimental import pallas as pl
from jax.experimental.pallas import tpu as pltpu
from jax.experimental.pallas import tpu_sc as plsc

_PREC = lax.Precision.HIGHEST


def _tca_body(x_ref, fmw_ref, fmb_ref, wxi_ref, wxj_ref, oh_ref, nemb_ref,
              wnti_ref, wntj_ref, h_ref, p_ref, q_ref):
    h = jnp.dot(x_ref[...], fmw_ref[...], preferred_element_type=jnp.float32,
                precision=_PREC) + fmb_ref[...]
    h_ref[...] = h
    ti = jnp.dot(nemb_ref[...], wnti_ref[...],
                 preferred_element_type=jnp.float32, precision=_PREC)
    tj = jnp.dot(nemb_ref[...], wntj_ref[...],
                 preferred_element_type=jnp.float32, precision=_PREC)
    oh = oh_ref[...]
    p_ref[...] = (jnp.dot(h, wxi_ref[...], preferred_element_type=jnp.float32,
                          precision=_PREC)
                  + jnp.dot(oh, ti, preferred_element_type=jnp.float32,
                            precision=_PREC))
    q_ref[...] = (jnp.dot(h, wxj_ref[...], preferred_element_type=jnp.float32,
                          precision=_PREC)
                  + jnp.dot(oh, tj, preferred_element_type=jnp.float32,
                            precision=_PREC))


def _tc_a(x, fm_w, fm_b, wxi, wxj, nt_oh, node_emb, wnti, wntj, blk):
    n, f = x.shape
    o = fm_w.shape[1]
    nd = node_emb.shape[1]
    grid = n // blk
    full = lambda i: (0, 0)
    outs = [jax.ShapeDtypeStruct((n, o), jnp.float32)] * 3
    return pl.pallas_call(
        _tca_body,
        grid=(grid,),
        in_specs=[
            pl.BlockSpec((blk, f), lambda i: (i, 0)),
            pl.BlockSpec((f, o), full),
            pl.BlockSpec((1, o), full),
            pl.BlockSpec((o, o), full),
            pl.BlockSpec((o, o), full),
            pl.BlockSpec((blk, 16), lambda i: (i, 0)),
            pl.BlockSpec((16, nd), full),
            pl.BlockSpec((nd, o), full),
            pl.BlockSpec((nd, o), full),
        ],
        out_specs=[pl.BlockSpec((blk, o), lambda i: (i, 0))] * 3,
        out_shape=outs,
    )(x, fm_w, fm_b.reshape(1, o), wxi, wxj, nt_oh, node_emb, wnti, wntj)


def _tcb_body(g1_ref, g2_ref, g3_ref, eoh_ref, ew_ref, eemb_ref, wet_ref,
              g1b_ref, g2w_ref, g2b_ref, out_ref):
    et_tab = jnp.dot(eemb_ref[...], wet_ref[...],
                     preferred_element_type=jnp.float32,
                     precision=_PREC) + g1b_ref[...]
    u = g1_ref[...] + g2_ref[...] + jnp.dot(
        eoh_ref[...], et_tab, preferred_element_type=jnp.float32,
        precision=_PREC)
    u = jnp.maximum(u, 0.0)
    t = jnp.sum(u * g2w_ref[...], axis=1, keepdims=True) + g2b_ref[...]
    s = jnp.maximum(t, 0.0) * ew_ref[...]
    m = g3_ref[...] * s
    half = m.shape[1] // 2
    out_ref[0] = m[:, :half]
    out_ref[1] = m[:, half:]


def _tc_b(g1, g2, g3, et_oh, ew, edge_emb, wet, g1b, g2w, g2b, blk):
    e, o = g1.shape
    ed = edge_emb.shape[1]
    grid = e // blk
    full = lambda i: (0, 0)
    return pl.pallas_call(
        _tcb_body,
        grid=(grid,),
        in_specs=[
            pl.BlockSpec((blk, o), lambda i: (i, 0)),
            pl.BlockSpec((blk, o), lambda i: (i, 0)),
            pl.BlockSpec((blk, o), lambda i: (i, 0)),
            pl.BlockSpec((blk, 16), lambda i: (i, 0)),
            pl.BlockSpec((blk, 1), lambda i: (i, 0)),
            pl.BlockSpec((16, ed), full),
            pl.BlockSpec((ed, o), full),
            pl.BlockSpec((1, o), full),
            pl.BlockSpec((1, o), full),
            pl.BlockSpec((1, 1), full),
        ],
        out_specs=[pl.BlockSpec((2, blk, o // 2), lambda i: (0, i, 0))],
        out_shape=[jax.ShapeDtypeStruct((2, e, o // 2), jnp.float32)],
    )(g1, g2, g3, et_oh, ew, edge_emb, wet, g1b.reshape(1, o),
      g2w.reshape(1, o), g2b.reshape(1, 1))[0]


def _tcc_body(h_ref, a_ref, out_ref):
    half = h_ref.shape[1] // 2
    v0 = h_ref[:, :half] + a_ref[0]
    v1 = h_ref[:, half:] + a_ref[1]
    out_ref[:, :half] = jnp.where(v0 >= 0.0, v0, 0.01 * v0)
    out_ref[:, half:] = jnp.where(v1 >= 0.0, v1, 0.01 * v1)


def _tc_c(h, aggr, blk):
    n, o = h.shape
    grid = n // blk
    return pl.pallas_call(
        _tcc_body,
        grid=(grid,),
        in_specs=[
            pl.BlockSpec((blk, o), lambda i: (i, 0)),
            pl.BlockSpec((2, blk, o // 2), lambda i: (0, i, 0)),
        ],
        out_specs=pl.BlockSpec((blk, o), lambda i: (i, 0)),
        out_shape=jax.ShapeDtypeStruct((n, o), jnp.float32),
    )(h, aggr)


def _sc_gather(p, q, h, dst2, src2):
    n, d = p.shape
    e = dst2.shape[1]
    w = 128
    nc = 2
    steps_per_core = e // w // nc
    mesh = plsc.VectorSubcoreMesh(core_axis_name="c", subcore_axis_name="s")
    outs = [jax.ShapeDtypeStruct((e, d), jnp.float32)] * 3

    @functools.partial(pl.kernel, mesh=mesh, out_type=outs, scratch_types=[])
    def k(p_hbm, q_hbm, h_hbm, dst_hbm, src_hbm, g1_hbm, g2_hbm, g3_hbm):
        idx_spec = pl.BlockSpec((1, w), lambda c, j: (0, c * steps_per_core + j))
        row_spec = pl.BlockSpec((w, d), lambda c, j: (c * steps_per_core + j, 0))

        def one_table(tbl_hbm, ihbm, ohbm):
            def body(i_v, o_v):
                pltpu.sync_copy(tbl_hbm.at[i_v.at[0]], o_v)

            pltpu.emit_pipeline(
                body,
                grid=(nc, steps_per_core),
                in_specs=[idx_spec],
                out_specs=[row_spec],
                core_axis_name=("c", "s"),
                dimension_semantics=(pltpu.PARALLEL, pltpu.PARALLEL),
            )(ihbm, ohbm)

        one_table(p_hbm, dst_hbm, g1_hbm)
        one_table(q_hbm, src_hbm, g2_hbm)
        one_table(h_hbm, src_hbm, g3_hbm)

    return k(p, q, h, dst2, src2)


def _sc_scatter(msg, dst2, zeros_half):
    _, e, d2 = msg.shape
    n = zeros_half.shape[0]
    w = 128
    steps = e // w
    ns = 16
    rows = n // ns
    r_lo = (rows // 8) * 8
    r_hi = n - r_lo * (ns - 1)
    mesh = plsc.VectorSubcoreMesh(core_axis_name="c", subcore_axis_name="s")

    @functools.partial(
        pl.kernel, mesh=mesh,
        out_type=jax.ShapeDtypeStruct((2, n, d2), jnp.float32),
        scratch_types=[pltpu.VMEM_SHARED((n, d2), jnp.float32)])
    def k(msg_hbm, dst_hbm, z_hbm, out_hbm, aggr_sh):
        cid = lax.axis_index("c")
        sid = lax.axis_index("s")

        @pl.when(sid < ns - 1)
        def _():
            pltpu.sync_copy(z_hbm.at[pl.ds(sid * r_lo, r_lo)],
                            aggr_sh.at[pl.ds(sid * r_lo, r_lo)])

        @pl.when(sid == ns - 1)
        def _():
            pltpu.sync_copy(z_hbm.at[pl.ds((ns - 1) * r_lo, r_hi)],
                            aggr_sh.at[pl.ds((ns - 1) * r_lo, r_hi)])

        plsc.subcore_barrier()

        def body(msg_v, idx_v):
            pltpu.sync_copy(msg_v, aggr_sh.at[idx_v.at[0]], add=True)

        pltpu.emit_pipeline(
            body,
            grid=(steps,),
            in_specs=[
                pl.BlockSpec((w, d2), lambda i: (i, 0)),
                pl.BlockSpec((1, w), lambda i: (0, i)),
            ],
            out_specs=[],
            core_axis_name="s",
            dimension_semantics=(pltpu.PARALLEL,),
        )(msg_hbm.at[cid], dst_hbm)

        plsc.subcore_barrier()

        @pl.when(sid < ns - 1)
        def _():
            pltpu.sync_copy(aggr_sh.at[pl.ds(sid * r_lo, r_lo)],
                            out_hbm.at[cid].at[pl.ds(sid * r_lo, r_lo)])

        @pl.when(sid == ns - 1)
        def _():
            pltpu.sync_copy(aggr_sh.at[pl.ds((ns - 1) * r_lo, r_hi)],
                            out_hbm.at[cid].at[pl.ds((ns - 1) * r_lo, r_hi)])

    return k(msg, dst2, zeros_half)


def kernel(x, edge_index, edge_attr, node_type, edge_type, node_emb, edge_emb,
           fm_w0, fm_b0, g1_w0, g1_b0, g2_w0, g2_b0,
           fm_w1, fm_b1, g1_w1, g1_b1, g2_w1, g2_b1):
    n, f = x.shape[1], x.shape[2]
    e = edge_index.shape[1]
    o = fm_w0.shape[1]
    nd = node_emb.shape[1]

    xs = x.reshape(n, f)
    src = edge_index[0]
    dst = edge_index[1]
    nt_oh = jax.nn.one_hot(node_type, 16, dtype=jnp.float32)
    et_oh = jax.nn.one_hot(edge_type, 16, dtype=jnp.float32)
    ew = edge_attr.reshape(e, 1)
    dst2 = dst.reshape(1, e)
    src2 = src.reshape(1, e)
    zeros_half = jnp.zeros((n, o // 2), jnp.float32)

    def layer(h_in, fm_w, fm_b, g1w, g1b, g2w, g2b):
        wxi = g1w[0:o]
        wxj = g1w[o:2 * o]
        wnti = g1w[2 * o:2 * o + nd]
        wntj = g1w[2 * o + nd:2 * o + 2 * nd]
        wet = g1w[2 * o + 2 * nd:]
        h, p, q = _tc_a(h_in, fm_w, fm_b, wxi, wxj, nt_oh, node_emb,
                        wnti, wntj, blk=1000)
        g1, g2, g3 = _sc_gather(p, q, h, dst2, src2)
        msg = _tc_b(g1, g2, g3, et_oh, ew, edge_emb, wet, g1b, g2w, g2b,
                    blk=2000)
        aggr = _sc_scatter(msg, dst2, zeros_half)
        return _tc_c(h, aggr, blk=1000)

    h1 = layer(xs, fm_w0, fm_b0, g1_w0, g1_b0, g2_w0, g2_b0)
    h2 = layer(h1, fm_w1, fm_b1, g1_w1, g1_b1, g2_w1, g2_b1)
    return h2.reshape(1, n, o)
